# baseline (speedup 1.0000x reference)
import functools
import math

import numpy as np
import jax
import jax.numpy as jnp
from jax import lax
from jax.experimental import pallas as pl
from jax.experimental.pallas import tpu as pltpu
from jax.experimental.pallas import tpu_sc as plsc

_GRID_SIZE = 5
_SPLINE_ORDER = 3
_COEFF = _GRID_SIZE + _SPLINE_ORDER
_NKNOT = _GRID_SIZE + 2 * _SPLINE_ORDER + 1

_KNOTS = (np.arange(-_SPLINE_ORDER, _GRID_SIZE + _SPLINE_ORDER + 1,
                    dtype=np.float32) * np.float32(2.0 / _GRID_SIZE)
          - np.float32(1.0))

_NC, _NS = 2, 16
_NW = _NC * _NS
_K = 128


def _spline_bases(x):
  t = _KNOTS
  bases = [((x >= t[j]) & (x < t[j + 1])).astype(x.dtype)
           for j in range(_NKNOT - 1)]
  for k in range(1, _SPLINE_ORDER + 1):
    nxt = []
    for j in range(_NKNOT - 1 - k):
      dl = np.float32(t[j + k] - t[j])
      dr = np.float32(t[j + k + 1] - t[j + 1])
      left = (x - t[j]) / dl
      right = (t[j + k + 1] - x) / dr
      nxt.append(left * bases[j] + right * bases[j + 1])
    bases = nxt
  return bases


def _features(h):
  return jnp.concatenate([h * jax.nn.sigmoid(h)] + _spline_bases(h), axis=1)


def _prep_weight(base_w, spline_w, d_pad=None):
  h_dim, d = base_w.shape
  d_pad = d if d_pad is None else d_pad
  groups = [base_w.T] + [spline_w[:, :, j].T for j in range(_COEFF)]
  groups = [jnp.pad(g, ((0, d_pad - d), (0, 0))) for g in groups]
  return jnp.concatenate(groups, axis=0)


@functools.partial(jax.jit, static_argnums=(4, 5, 6))
def _sc_agg(x_pad, src_p, dst_p, zeros_sh, n_chunks, d, tiled=True):
  n_pad = x_pad.shape[0]
  sh_rows = zeros_sh.shape[0]
  rows_per_sub = sh_rows // _NS
  out_rows = n_pad // _NS
  mesh = plsc.VectorSubcoreMesh(core_axis_name="c", subcore_axis_name="s",
                                num_cores=_NC, num_subcores=_NS)

  @functools.partial(
      pl.kernel,
      out_type=jax.ShapeDtypeStruct((_NC, n_pad, d), jnp.float32),
      mesh=mesh,
      scratch_types=[
          pltpu.VMEM((n_chunks, _K), jnp.int32),
          pltpu.VMEM((_K,), jnp.int32),
          pltpu.VMEM((_K,), jnp.int32),
          pltpu.VMEM((_K, d), jnp.float32),
          pltpu.VMEM((_K, d), jnp.float32),
          pltpu.VMEM_SHARED((sh_rows, d), jnp.float32),
          pltpu.SemaphoreType.DMA,
          pltpu.SemaphoreType.DMA,
          pltpu.SemaphoreType.DMA,
          pltpu.SemaphoreType.DMA,
      ],
      compiler_params=pltpu.CompilerParams(use_tc_tiling_on_sc=tiled),
  )
  def agg(x_hbm, src_hbm, dst_hbm, zero_hbm, out_hbm,
          dst_v, sidx_a, sidx_b, rows_a, rows_b, acc_sh,
          sem_a, sem_b, sem_ia, sem_ib):
    c = lax.axis_index("c")
    s = lax.axis_index("s")
    wid = c * _NS + s
    pltpu.sync_copy(zero_hbm.at[pl.ds(s * rows_per_sub, rows_per_sub)],
                    acc_sh.at[pl.ds(s * rows_per_sub, rows_per_sub)])
    pltpu.sync_copy(dst_hbm.at[wid], dst_v)
    pltpu.sync_copy(src_hbm.at[wid, 0], sidx_a)
    pltpu.sync_copy(src_hbm.at[wid, 1], sidx_b)
    plsc.subcore_barrier()

    def pair(p, carry):
      i0 = 2 * p
      i1 = i0 + 1
      d0 = pltpu.async_copy(x_hbm.at[sidx_a], rows_a, sem_a)
      d1 = pltpu.async_copy(x_hbm.at[sidx_b], rows_b, sem_b)
      d0.wait()
      la = pltpu.async_copy(src_hbm.at[wid, lax.rem(i0 + 2, n_chunks)],
                            sidx_a, sem_ia)
      pltpu.sync_copy(rows_a, acc_sh.at[dst_v.at[i0]], add=True)
      d1.wait()
      lb = pltpu.async_copy(src_hbm.at[wid, lax.rem(i1 + 2, n_chunks)],
                            sidx_b, sem_ib)
      pltpu.sync_copy(rows_b, acc_sh.at[dst_v.at[i1]], add=True)
      la.wait()
      lb.wait()
      return carry

    lax.fori_loop(0, n_chunks // 2, pair, 0)
    plsc.subcore_barrier()
    pltpu.sync_copy(acc_sh.at[pl.ds(s * out_rows, out_rows)],
                    out_hbm.at[c, pl.ds(s * out_rows, out_rows)])

  return agg(x_pad, src_p, dst_p, zeros_sh)


def _kan_layer(x_pad, aggs, w, blk, out_cols=None):
  n_pad, d = x_pad.shape
  h_dim = w.shape[1]
  out_cols = h_dim if out_cols is None else out_cols

  def body(x_ref, a_ref, w_ref, o_ref):
    h = x_ref[...] + a_ref[0] + a_ref[1]
    acc = jnp.dot(_features(h), w_ref[...], preferred_element_type=jnp.float32)
    if out_cols > h_dim:
      acc = jnp.concatenate(
          [acc, jnp.zeros((blk, out_cols - h_dim), jnp.float32)], axis=1)
    o_ref[...] = acc

  return pl.pallas_call(
      body,
      grid=(n_pad // blk,),
      in_specs=[
          pl.BlockSpec((blk, d), lambda i: (i, 0)),
          pl.BlockSpec((2, blk, d), lambda i: (0, i, 0)),
          pl.BlockSpec((9 * d, h_dim), lambda i: (0, 0)),
      ],
      out_specs=pl.BlockSpec((blk, out_cols), lambda i: (i, 0)),
      out_shape=jax.ShapeDtypeStruct((n_pad, out_cols), jnp.float32),
  )(x_pad, aggs, w)


def _kan_layer_pool(x_pad, aggs, w, batch2d, n_graphs, blk):
  n_pad, d = x_pad.shape
  h_dim = w.shape[1]

  def body(x_ref, a_ref, w_ref, b_ref, pool_ref):
    h = x_ref[...] + a_ref[0] + a_ref[1]
    hid = jnp.dot(_features(h), w_ref[...], preferred_element_type=jnp.float32)
    seg = lax.broadcasted_iota(jnp.int32, (blk, n_graphs), 1)
    m = (b_ref[...] == seg).astype(jnp.float32)
    contrib = lax.dot_general(m, hid, (((0,), (0,)), ((), ())),
                              preferred_element_type=jnp.float32)
    i = pl.program_id(0)

    @pl.when(i == 0)
    def _():
      pool_ref[...] = contrib

    @pl.when(i > 0)
    def _():
      pool_ref[...] += contrib

  return pl.pallas_call(
      body,
      grid=(n_pad // blk,),
      in_specs=[
          pl.BlockSpec((blk, d), lambda i: (i, 0)),
          pl.BlockSpec((2, blk, d), lambda i: (0, i, 0)),
          pl.BlockSpec((9 * d, h_dim), lambda i: (0, 0)),
          pl.BlockSpec((blk, 1), lambda i: (i, 0)),
      ],
      out_specs=pl.BlockSpec((n_graphs, h_dim), lambda i: (0, 0)),
      out_shape=jax.ShapeDtypeStruct((n_graphs, h_dim), jnp.float32),
  )(x_pad, aggs, w, batch2d)


def _kan_final(pooled, w):
  g, h_dim = pooled.shape
  tp = w.shape[1]

  def body(x_ref, w_ref, o_ref):
    o_ref[...] = jnp.dot(_features(x_ref[...]), w_ref[...],
                         preferred_element_type=jnp.float32)

  return pl.pallas_call(
      body,
      out_shape=jax.ShapeDtypeStruct((g, tp), jnp.float32),
  )(pooled, w)


def kernel(x, edge_index, batch, conv0_base_w, conv0_spline_w,
           conv1_base_w, conv1_spline_w, kan_base_w, kan_spline_w):
  n, d = x.shape
  h_dim = conv0_base_w.shape[0]
  t_dim = kan_base_w.shape[0]
  n_graphs = 128
  e = edge_index.shape[1]

  blk = 256
  n_pad = math.ceil(n / blk) * blk
  sh_rows = n_pad + 128

  n_chunks = math.ceil(e / (_NW * _K))
  n_chunks += n_chunks % 2
  e_pad = _NW * _K * n_chunks
  src = jnp.concatenate(
      [edge_index[0], jnp.zeros((e_pad - e,), jnp.int32)]).reshape(
          _NW, n_chunks, _K)
  pad_dst = n_pad + (jnp.arange(e_pad - e, dtype=jnp.int32) % 128)
  dst = jnp.concatenate([edge_index[1], pad_dst]).reshape(_NW, n_chunks, _K)

  x_pad = jnp.pad(x, ((0, n_pad - n), (0, 0)))
  batch2d = jnp.pad(batch, (0, n_pad - n),
                    constant_values=n_graphs).reshape(n_pad, 1)

  h_pad = 128
  w0 = _prep_weight(conv0_base_w, conv0_spline_w)
  w1 = _prep_weight(conv1_base_w, conv1_spline_w, h_pad)
  tp = 128
  w2 = jnp.pad(_prep_weight(kan_base_w, kan_spline_w),
               ((0, 0), (0, tp - t_dim)))

  zeros_d = jnp.zeros((sh_rows, d), jnp.float32)

  agg0 = _sc_agg(x_pad, src, dst, zeros_d, n_chunks, d)
  h1 = _kan_layer(x_pad, agg0, w0, blk, out_cols=h_pad)
  agg1 = _sc_agg(h1, src, dst, zeros_d, n_chunks, h_pad)
  pooled = _kan_layer_pool(h1, agg1, w1, batch2d, n_graphs, blk)
  out = _kan_final(pooled, w2)
  return out[:, :t_dim]


# speedup vs baseline: 1.6607x; 1.6607x over previous
import functools
import math

import numpy as np
import jax
import jax.numpy as jnp
from jax import lax
from jax.experimental import pallas as pl
from jax.experimental.pallas import tpu as pltpu
from jax.experimental.pallas import tpu_sc as plsc

_GRID_SIZE = 5
_SPLINE_ORDER = 3
_COEFF = _GRID_SIZE + _SPLINE_ORDER
_NKNOT = _GRID_SIZE + 2 * _SPLINE_ORDER + 1

_KNOTS = (np.arange(-_SPLINE_ORDER, _GRID_SIZE + _SPLINE_ORDER + 1,
                    dtype=np.float32) * np.float32(2.0 / _GRID_SIZE)
          - np.float32(1.0))

_NC, _NS = 2, 16
_NW = _NC * _NS
_K = 128


def _spline_bases(x):
  t = _KNOTS
  bases = [((x >= t[j]) & (x < t[j + 1])).astype(x.dtype)
           for j in range(_NKNOT - 1)]
  for k in range(1, _SPLINE_ORDER + 1):
    nxt = []
    for j in range(_NKNOT - 1 - k):
      dl = np.float32(t[j + k] - t[j])
      dr = np.float32(t[j + k + 1] - t[j + 1])
      left = (x - t[j]) / dl
      right = (t[j + k + 1] - x) / dr
      nxt.append(left * bases[j] + right * bases[j + 1])
    bases = nxt
  return bases


def _features(h):
  return jnp.concatenate([h * jax.nn.sigmoid(h)] + _spline_bases(h), axis=1)


def _prep_weight(base_w, spline_w, d_pad=None):
  h_dim, d = base_w.shape
  d_pad = d if d_pad is None else d_pad
  groups = [base_w.T] + [spline_w[:, :, j].T for j in range(_COEFF)]
  groups = [jnp.pad(g, ((0, d_pad - d), (0, 0))) for g in groups]
  return jnp.concatenate(groups, axis=0)


@functools.partial(jax.jit, static_argnums=(4, 5, 6))
def _sc_agg(x_pad, src_p, dst_p, zeros_sh, n_chunks, d, tiled=True):
  n_pad = x_pad.shape[0]
  sh_rows = zeros_sh.shape[0]
  rows_per_sub = sh_rows // _NS
  out_rows = n_pad // _NS
  mesh = plsc.VectorSubcoreMesh(core_axis_name="c", subcore_axis_name="s",
                                num_cores=_NC, num_subcores=_NS)

  @functools.partial(
      pl.kernel,
      out_type=jax.ShapeDtypeStruct((_NC, n_pad, d), jnp.float32),
      mesh=mesh,
      scratch_types=[
          pltpu.VMEM((n_chunks, _K), jnp.int32),
          pltpu.VMEM((n_chunks, _K), jnp.int32),
          pltpu.VMEM((_K, d), jnp.float32),
          pltpu.VMEM_SHARED((sh_rows, d), jnp.float32),
          pltpu.SemaphoreType.DMA,
      ],
      compiler_params=pltpu.CompilerParams(use_tc_tiling_on_sc=tiled),
  )
  def agg(x_hbm, src_hbm, dst_hbm, zero_hbm, out_hbm,
          src_v, dst_v, rows_v, acc_sh, sem):
    c = lax.axis_index("c")
    s = lax.axis_index("s")
    wid = c * _NS + s
    pltpu.sync_copy(zero_hbm.at[pl.ds(s * rows_per_sub, rows_per_sub)],
                    acc_sh.at[pl.ds(s * rows_per_sub, rows_per_sub)])
    pltpu.sync_copy(src_hbm.at[wid], src_v)
    pltpu.sync_copy(dst_hbm.at[wid], dst_v)
    plsc.subcore_barrier()

    def chunk(i, carry):
      pltpu.async_copy(x_hbm.at[src_v.at[i]], rows_v, sem).wait()
      pltpu.sync_copy(rows_v, acc_sh.at[dst_v.at[i]], add=True)
      return carry

    lax.fori_loop(0, n_chunks, chunk, 0)
    plsc.subcore_barrier()
    pltpu.sync_copy(acc_sh.at[pl.ds(s * out_rows, out_rows)],
                    out_hbm.at[c, pl.ds(s * out_rows, out_rows)])

  return agg(x_pad, src_p, dst_p, zeros_sh)


def _kan_layer(x_pad, aggs, w, blk, out_cols=None):
  n_pad, d = x_pad.shape
  h_dim = w.shape[1]
  out_cols = h_dim if out_cols is None else out_cols

  def body(x_ref, a_ref, w_ref, o_ref):
    h = x_ref[...] + a_ref[0] + a_ref[1]
    acc = jnp.dot(_features(h), w_ref[...], preferred_element_type=jnp.float32)
    if out_cols > h_dim:
      acc = jnp.concatenate(
          [acc, jnp.zeros((blk, out_cols - h_dim), jnp.float32)], axis=1)
    o_ref[...] = acc

  return pl.pallas_call(
      body,
      grid=(n_pad // blk,),
      in_specs=[
          pl.BlockSpec((blk, d), lambda i: (i, 0)),
          pl.BlockSpec((2, blk, d), lambda i: (0, i, 0)),
          pl.BlockSpec((9 * d, h_dim), lambda i: (0, 0)),
      ],
      out_specs=pl.BlockSpec((blk, out_cols), lambda i: (i, 0)),
      out_shape=jax.ShapeDtypeStruct((n_pad, out_cols), jnp.float32),
  )(x_pad, aggs, w)


def _kan_layer_pool(x_pad, aggs, w, batch2d, n_graphs, blk):
  n_pad, d = x_pad.shape
  h_dim = w.shape[1]

  def body(x_ref, a_ref, w_ref, b_ref, pool_ref):
    h = x_ref[...] + a_ref[0] + a_ref[1]
    hid = jnp.dot(_features(h), w_ref[...], preferred_element_type=jnp.float32)
    seg = lax.broadcasted_iota(jnp.int32, (blk, n_graphs), 1)
    m = (b_ref[...] == seg).astype(jnp.float32)
    contrib = lax.dot_general(m, hid, (((0,), (0,)), ((), ())),
                              preferred_element_type=jnp.float32)
    i = pl.program_id(0)

    @pl.when(i == 0)
    def _():
      pool_ref[...] = contrib

    @pl.when(i > 0)
    def _():
      pool_ref[...] += contrib

  return pl.pallas_call(
      body,
      grid=(n_pad // blk,),
      in_specs=[
          pl.BlockSpec((blk, d), lambda i: (i, 0)),
          pl.BlockSpec((2, blk, d), lambda i: (0, i, 0)),
          pl.BlockSpec((9 * d, h_dim), lambda i: (0, 0)),
          pl.BlockSpec((blk, 1), lambda i: (i, 0)),
      ],
      out_specs=pl.BlockSpec((n_graphs, h_dim), lambda i: (0, 0)),
      out_shape=jax.ShapeDtypeStruct((n_graphs, h_dim), jnp.float32),
  )(x_pad, aggs, w, batch2d)


def _kan_final(pooled, w):
  g, h_dim = pooled.shape
  tp = w.shape[1]

  def body(x_ref, w_ref, o_ref):
    o_ref[...] = jnp.dot(_features(x_ref[...]), w_ref[...],
                         preferred_element_type=jnp.float32)

  return pl.pallas_call(
      body,
      out_shape=jax.ShapeDtypeStruct((g, tp), jnp.float32),
  )(pooled, w)


def kernel(x, edge_index, batch, conv0_base_w, conv0_spline_w,
           conv1_base_w, conv1_spline_w, kan_base_w, kan_spline_w):
  n, d = x.shape
  h_dim = conv0_base_w.shape[0]
  t_dim = kan_base_w.shape[0]
  n_graphs = 128
  e = edge_index.shape[1]

  blk = 256
  n_pad = math.ceil(n / blk) * blk
  sh_rows = n_pad + 128

  n_chunks = math.ceil(e / (_NW * _K))
  e_pad = _NW * _K * n_chunks
  src = jnp.concatenate(
      [edge_index[0], jnp.zeros((e_pad - e,), jnp.int32)]).reshape(
          _NW, n_chunks, _K)
  pad_dst = n_pad + (jnp.arange(e_pad - e, dtype=jnp.int32) % 128)
  dst = jnp.concatenate([edge_index[1], pad_dst]).reshape(_NW, n_chunks, _K)

  x_pad = jnp.pad(x, ((0, n_pad - n), (0, 0)))
  batch2d = jnp.pad(batch, (0, n_pad - n),
                    constant_values=n_graphs).reshape(n_pad, 1)

  w0 = _prep_weight(conv0_base_w, conv0_spline_w)
  w1 = _prep_weight(conv1_base_w, conv1_spline_w)
  tp = 128
  w2 = jnp.pad(_prep_weight(kan_base_w, kan_spline_w),
               ((0, 0), (0, tp - t_dim)))

  zeros_d = jnp.zeros((sh_rows, d), jnp.float32)
  zeros_h = jnp.zeros((sh_rows, h_dim), jnp.float32)

  agg0 = _sc_agg(x_pad, src, dst, zeros_d, n_chunks, d)
  h1 = _kan_layer(x_pad, agg0, w0, blk)
  agg1 = _sc_agg(h1, src, dst, zeros_h, n_chunks, h_dim, False)
  pooled = _kan_layer_pool(h1, agg1, w1, batch2d, n_graphs, blk)
  out = _kan_final(pooled, w2)
  return out[:, :t_dim]
